# Initial kernel scaffold; baseline (speedup 1.0000x reference)
import jax
import jax.numpy as jnp
from jax.experimental import pallas as pl


def kernel(adj_edge_index, adj_edge_weight, subseq_emb, target_emb):
    raise NotImplementedError("write your pallas kernel here")


# speedup vs baseline: 9.4552x; 9.4552x over previous
import jax
import jax.numpy as jnp
from jax import lax
from jax.experimental import pallas as pl
from jax.experimental.pallas import tpu as pltpu
from jax.experimental.pallas import tpu_sc as plsc

N_NODES = 10000
N_PAD = 10240
N_SUBSEQ = 6000
D = 128
DH = D // 2
N_EDGES_RAW = 320000
LAYERS = 3

NC = 2
NS = 16
CHUNK = 128
CHUNKS_PER_TILE = 160
EDGES_PER_TILE = CHUNK * CHUNKS_PER_TILE
E_PAD = EDGES_PER_TILE * NS
TOTAL_CHUNKS = E_PAD // CHUNK

ROWS_PER_TILE = N_PAD // NS
WB = 128
WB_STEPS = ROWS_PER_TILE // WB


def _gcn_body(src_hbm, dst_hbm, w_hbm, ini_st, o1_st, o2_st, o3_st,
              acc_sh, srcb, dstb, wbuf, rows0, rows1,
              gsem0, gsem1, ssem0, ssem1):
    c = lax.axis_index("c")
    s = lax.axis_index("s")
    rows = (rows0, rows1)
    gsem = (gsem0, gsem1)
    ssem = (ssem0, ssem1)
    tables = (ini_st, o1_st, o2_st)
    outs = (o1_st, o2_st, o3_st)

    chunk0 = s * CHUNKS_PER_TILE
    pltpu.sync_copy(src_hbm.at[pl.ds(chunk0, CHUNKS_PER_TILE)], srcb)
    pltpu.sync_copy(dst_hbm.at[pl.ds(chunk0, CHUNKS_PER_TILE)], dstb)
    pltpu.sync_copy(w_hbm.at[pl.ds(chunk0, CHUNKS_PER_TILE)], wbuf)

    zero16 = jnp.zeros((16,), jnp.float32)

    for layer in range(LAYERS):
        tbl = tables[layer].at[c]
        out = outs[layer].at[c]

        def zrow(r, carry):
            for f in range(DH // 16):
                rows0[r, pl.ds(f * 16, 16)] = zero16
            return carry

        lax.fori_loop(0, WB, zrow, 0)
        for b in range(WB_STEPS):
            pltpu.sync_copy(
                rows0, acc_sh.at[pl.ds(s * ROWS_PER_TILE + b * WB, WB)])
        plsc.subcore_barrier()

        def start_gather(ci, b):
            pltpu.async_copy(tbl.at[srcb.at[ci]], rows[b], gsem[b])

        def wait_gather(ci, b):
            pltpu.make_async_copy(tbl.at[srcb.at[ci]], rows[b], gsem[b]).wait()

        def start_scatter(ci, b):
            pltpu.async_copy(rows[b], acc_sh.at[dstb.at[ci]], ssem[b], add=True)

        def wait_scatter(ci, b):
            pltpu.make_async_copy(rows[b], acc_sh.at[dstb.at[ci]], ssem[b]).wait()

        start_gather(0, 0)

        def step(ci, b, nb):
            @pl.when(ci >= 1)
            def _():
                wait_scatter(ci - 1, nb)

            @pl.when(ci < CHUNKS_PER_TILE - 1)
            def _():
                start_gather(ci + 1, nb)

            wait_gather(ci, b)
            rbuf = rows[b]

            @plsc.parallel_loop(0, CHUNK // 16, unroll=2)
            def grp(g):
                w16 = wbuf[ci, pl.ds(g * 16, 16)]
                for l in range(16):
                    e = g * 16 + l
                    wv = jnp.full((16,), w16[l], jnp.float32)
                    for f in range(DH // 16):
                        sl = pl.ds(f * 16, 16)
                        rbuf[e, sl] = rbuf[e, sl] * wv

            start_scatter(ci, b)

        def outer(oi, carry):
            step(oi * 2, 0, 1)
            step(oi * 2 + 1, 1, 0)
            return carry

        lax.fori_loop(0, CHUNKS_PER_TILE // 2, outer, 0)
        wait_scatter(CHUNKS_PER_TILE - 1, 1)
        plsc.subcore_barrier()

        sl = pl.ds(s * ROWS_PER_TILE, ROWS_PER_TILE)
        pltpu.sync_copy(acc_sh.at[sl], out.at[sl])
        plsc.subcore_barrier()


@jax.jit
def _propagate(src, dst, w, ini_st):
    mesh = plsc.VectorSubcoreMesh(core_axis_name="c", subcore_axis_name="s")
    gcn = pl.kernel(
        _gcn_body,
        out_type=(
            jax.ShapeDtypeStruct((NC, N_PAD, DH), jnp.float32),
            jax.ShapeDtypeStruct((NC, N_PAD, DH), jnp.float32),
            jax.ShapeDtypeStruct((NC, N_PAD, DH), jnp.float32),
        ),
        mesh=mesh,
        compiler_params=pltpu.CompilerParams(use_tc_tiling_on_sc=False),
        scratch_types=[
            pltpu.VMEM_SHARED((N_PAD, DH), jnp.float32),
            pltpu.VMEM((CHUNKS_PER_TILE, CHUNK), jnp.int32),
            pltpu.VMEM((CHUNKS_PER_TILE, CHUNK), jnp.int32),
            pltpu.VMEM((CHUNKS_PER_TILE, CHUNK), jnp.float32),
            pltpu.VMEM((CHUNK, DH), jnp.float32),
            pltpu.VMEM((CHUNK, DH), jnp.float32),
            pltpu.SemaphoreType.DMA,
            pltpu.SemaphoreType.DMA,
            pltpu.SemaphoreType.DMA,
            pltpu.SemaphoreType.DMA,
        ],
    )
    o1, o2, o3 = gcn(src, dst, w, ini_st)
    total = ini_st + o1 + o2 + o3
    sum_emb = jnp.concatenate([total[0], total[1]], axis=1) / (LAYERS + 1)
    return sum_emb


def kernel(adj_edge_index, adj_edge_weight, subseq_emb, target_emb):
    src = adj_edge_index[0]
    dst = adj_edge_index[1]
    pad = E_PAD - N_EDGES_RAW
    pad_idx = (jnp.arange(pad, dtype=jnp.int32) * 37) % N_NODES
    src = jnp.concatenate([src, pad_idx]).reshape(TOTAL_CHUNKS, CHUNK)
    dst = jnp.concatenate([dst, pad_idx]).reshape(TOTAL_CHUNKS, CHUNK)
    w = jnp.concatenate(
        [adj_edge_weight, jnp.zeros((pad,), jnp.float32)]
    ).reshape(TOTAL_CHUNKS, CHUNK)
    ini = jnp.concatenate(
        [subseq_emb, target_emb, jnp.zeros((N_PAD - N_NODES, D), jnp.float32)],
        axis=0)
    ini_st = jnp.stack([ini[:, :DH], ini[:, DH:]], axis=0)
    sum_emb = _propagate(src, dst, w, ini_st)[:N_NODES]
    return (sum_emb[:N_SUBSEQ], sum_emb[N_SUBSEQ:])


# speedup vs baseline: 10.7160x; 1.1333x over previous
import jax
import jax.numpy as jnp
from jax import lax
from jax.experimental import pallas as pl
from jax.experimental.pallas import tpu as pltpu
from jax.experimental.pallas import tpu_sc as plsc

N_NODES = 10000
N_PAD = 10240
N_SUBSEQ = 6000
D = 128
DH = D // 2
N_EDGES_RAW = 320000
LAYERS = 3

NC = 2
NS = 16
CHUNK = 128
CHUNKS_PER_TILE = 160
EDGES_PER_TILE = CHUNK * CHUNKS_PER_TILE
E_PAD = EDGES_PER_TILE * NS
TOTAL_CHUNKS = E_PAD // CHUNK

ROWS_PER_TILE = N_PAD // NS
WB = 128
WB_STEPS = ROWS_PER_TILE // WB


def _gcn_body(src_hbm, dst_hbm, w_hbm, ini_st, o1_st, o2_st, o3_st,
              acc_sh, srcb, dstb, wbuf, rows0, rows1, rows2,
              gsem0, gsem1, gsem2, ssem0, ssem1, ssem2):
    c = lax.axis_index("c")
    s = lax.axis_index("s")
    rows = (rows0, rows1, rows2)
    gsem = (gsem0, gsem1, gsem2)
    ssem = (ssem0, ssem1, ssem2)
    tables = (ini_st, o1_st, o2_st)
    outs = (o1_st, o2_st, o3_st)

    chunk0 = s * CHUNKS_PER_TILE
    pltpu.sync_copy(src_hbm.at[pl.ds(chunk0, CHUNKS_PER_TILE)], srcb)
    pltpu.sync_copy(dst_hbm.at[pl.ds(chunk0, CHUNKS_PER_TILE)], dstb)
    pltpu.sync_copy(w_hbm.at[pl.ds(chunk0, CHUNKS_PER_TILE)], wbuf)

    zero16 = jnp.zeros((16,), jnp.float32)

    for layer in range(LAYERS):
        tbl = tables[layer].at[c]
        out = outs[layer].at[c]

        def zrow(r, carry):
            for f in range(DH // 16):
                rows0[r, pl.ds(f * 16, 16)] = zero16
            return carry

        lax.fori_loop(0, WB, zrow, 0)
        for b in range(WB_STEPS):
            pltpu.sync_copy(
                rows0, acc_sh.at[pl.ds(s * ROWS_PER_TILE + b * WB, WB)])
        plsc.subcore_barrier()

        def start_gather(ci, b):
            pltpu.async_copy(tbl.at[srcb.at[ci]], rows[b], gsem[b])

        def wait_gather(ci, b):
            pltpu.make_async_copy(tbl.at[srcb.at[ci]], rows[b], gsem[b]).wait()

        def start_scatter(ci, b):
            pltpu.async_copy(rows[b], acc_sh.at[dstb.at[ci]], ssem[b], add=True)

        def wait_scatter(ci, b):
            pltpu.make_async_copy(rows[b], acc_sh.at[dstb.at[ci]], ssem[b]).wait()

        start_gather(0, 0)

        def step(ci, b, nb):
            @pl.when(ci >= 2)
            def _():
                wait_scatter(ci - 2, nb)

            @pl.when(ci < CHUNKS_PER_TILE - 1)
            def _():
                start_gather(ci + 1, nb)

            wait_gather(ci, b)
            rbuf = rows[b]

            @plsc.parallel_loop(0, CHUNK // 16, unroll=2)
            def grp(g):
                w16 = wbuf[ci, pl.ds(g * 16, 16)]
                for l in range(16):
                    e = g * 16 + l
                    wv = jnp.full((16,), w16[l], jnp.float32)
                    for f in range(DH // 16):
                        sl = pl.ds(f * 16, 16)
                        rbuf[e, sl] = rbuf[e, sl] * wv

            start_scatter(ci, b)

        step(0, 0, 1)

        def outer(oi, carry):
            step(oi * 3 + 1, 1, 2)
            step(oi * 3 + 2, 2, 0)
            step(oi * 3 + 3, 0, 1)
            return carry

        lax.fori_loop(0, (CHUNKS_PER_TILE - 1) // 3, outer, 0)
        wait_scatter(CHUNKS_PER_TILE - 2, (CHUNKS_PER_TILE - 2) % 3)
        wait_scatter(CHUNKS_PER_TILE - 1, (CHUNKS_PER_TILE - 1) % 3)
        plsc.subcore_barrier()

        sl = pl.ds(s * ROWS_PER_TILE, ROWS_PER_TILE)
        pltpu.sync_copy(acc_sh.at[sl], out.at[sl])
        plsc.subcore_barrier()


@jax.jit
def _propagate(src, dst, w, ini_st):
    mesh = plsc.VectorSubcoreMesh(core_axis_name="c", subcore_axis_name="s")
    gcn = pl.kernel(
        _gcn_body,
        out_type=(
            jax.ShapeDtypeStruct((NC, N_PAD, DH), jnp.float32),
            jax.ShapeDtypeStruct((NC, N_PAD, DH), jnp.float32),
            jax.ShapeDtypeStruct((NC, N_PAD, DH), jnp.float32),
        ),
        mesh=mesh,
        compiler_params=pltpu.CompilerParams(use_tc_tiling_on_sc=False),
        scratch_types=[
            pltpu.VMEM_SHARED((N_PAD, DH), jnp.float32),
            pltpu.VMEM((CHUNKS_PER_TILE, CHUNK), jnp.int32),
            pltpu.VMEM((CHUNKS_PER_TILE, CHUNK), jnp.int32),
            pltpu.VMEM((CHUNKS_PER_TILE, CHUNK), jnp.float32),
            pltpu.VMEM((CHUNK, DH), jnp.float32),
            pltpu.VMEM((CHUNK, DH), jnp.float32),
            pltpu.VMEM((CHUNK, DH), jnp.float32),
            pltpu.SemaphoreType.DMA,
            pltpu.SemaphoreType.DMA,
            pltpu.SemaphoreType.DMA,
            pltpu.SemaphoreType.DMA,
            pltpu.SemaphoreType.DMA,
            pltpu.SemaphoreType.DMA,
        ],
    )
    o1, o2, o3 = gcn(src, dst, w, ini_st)
    total = ini_st + o1 + o2 + o3
    sum_emb = jnp.concatenate([total[0], total[1]], axis=1) / (LAYERS + 1)
    return sum_emb


def kernel(adj_edge_index, adj_edge_weight, subseq_emb, target_emb):
    src = adj_edge_index[0]
    dst = adj_edge_index[1]
    pad = E_PAD - N_EDGES_RAW
    pad_idx = (jnp.arange(pad, dtype=jnp.int32) * 37) % N_NODES
    src = jnp.concatenate([src, pad_idx]).reshape(TOTAL_CHUNKS, CHUNK)
    dst = jnp.concatenate([dst, pad_idx]).reshape(TOTAL_CHUNKS, CHUNK)
    w = jnp.concatenate(
        [adj_edge_weight, jnp.zeros((pad,), jnp.float32)]
    ).reshape(TOTAL_CHUNKS, CHUNK)
    ini = jnp.concatenate(
        [subseq_emb, target_emb, jnp.zeros((N_PAD - N_NODES, D), jnp.float32)],
        axis=0)
    ini_st = jnp.stack([ini[:, :DH], ini[:, DH:]], axis=0)
    sum_emb = _propagate(src, dst, w, ini_st)[:N_NODES]
    return (sum_emb[:N_SUBSEQ], sum_emb[N_SUBSEQ:])


# speedup vs baseline: 11.4904x; 1.0723x over previous
import jax
import jax.numpy as jnp
from jax import lax
from jax.experimental import pallas as pl
from jax.experimental.pallas import tpu as pltpu
from jax.experimental.pallas import tpu_sc as plsc

N_NODES = 10000
N_PAD = 10240
N_SUBSEQ = 6000
D = 128
DH = D // 2
N_EDGES_RAW = 320000
LAYERS = 3

NC = 2
NS = 16
CHUNK = 128
CHUNKS_PER_TILE = 160
EDGES_PER_TILE = CHUNK * CHUNKS_PER_TILE
E_PAD = EDGES_PER_TILE * NS
TOTAL_CHUNKS = E_PAD // CHUNK

ROWS_PER_TILE = N_PAD // NS
WB = 128
WB_STEPS = ROWS_PER_TILE // WB


SRCN = 8
SRC_AHEAD = 5


def _gcn_body(src_hbm, dst_hbm, w_hbm, ini_st, o1_st, o2_st, o3_st,
              acc_sh, dstb, wbuf, rows0, rows1, rows2, rows3, sbufs,
              gsem0, gsem1, gsem2, gsem3, ssem0, ssem1, ssem2, ssem3,
              srcsems):
    c = lax.axis_index("c")
    s = lax.axis_index("s")
    rows = (rows0, rows1, rows2, rows3)
    gsem = (gsem0, gsem1, gsem2, gsem3)
    ssem = (ssem0, ssem1, ssem2, ssem3)
    tables = (ini_st, o1_st, o2_st)
    outs = (o1_st, o2_st, o3_st)

    chunk0 = s * CHUNKS_PER_TILE
    pltpu.sync_copy(dst_hbm.at[pl.ds(chunk0, CHUNKS_PER_TILE)], dstb)
    pltpu.sync_copy(w_hbm.at[pl.ds(chunk0, CHUNKS_PER_TILE)], wbuf)

    zero16 = jnp.zeros((16,), jnp.float32)

    for layer in range(LAYERS):
        tbl = tables[layer].at[c]
        out = outs[layer].at[c]

        def zrow(r, carry):
            for f in range(DH // 16):
                rows0[r, pl.ds(f * 16, 16)] = zero16
            return carry

        lax.fori_loop(0, WB, zrow, 0)
        for b in range(WB_STEPS):
            pltpu.sync_copy(
                rows0, acc_sh.at[pl.ds(s * ROWS_PER_TILE + b * WB, WB)])
        plsc.subcore_barrier()

        def start_scatter(ci, b):
            pltpu.async_copy(rows[b], acc_sh.at[dstb.at[ci]], ssem[b], add=True)

        def wait_scatter(ci, b):
            pltpu.make_async_copy(rows[b], acc_sh.at[dstb.at[ci]], ssem[b]).wait()

        def start_srcload(ci):
            j = ci % SRCN
            pltpu.async_copy(src_hbm.at[chunk0 + ci], sbufs.at[j],
                             srcsems.at[j])

        def wait_srcload(ci):
            j = ci % SRCN
            pltpu.make_async_copy(src_hbm.at[chunk0 + ci], sbufs.at[j],
                                  srcsems.at[j]).wait()

        def start_gather(ci, b):
            pltpu.async_copy(tbl.at[sbufs.at[ci % SRCN]], rows[b], gsem[b])

        def wait_gather(ci, b):
            pltpu.make_async_copy(
                tbl.at[sbufs.at[ci % SRCN]], rows[b], gsem[b]).wait()

        for j in range(SRC_AHEAD):
            start_srcload(j)
        wait_srcload(0)
        start_gather(0, 0)
        wait_srcload(1)
        start_gather(1, 1)

        def step(ci, b, nb):
            @pl.when(ci >= 2)
            def _():
                wait_scatter(ci - 2, nb)

            @pl.when(ci < CHUNKS_PER_TILE - SRC_AHEAD)
            def _():
                start_srcload(ci + SRC_AHEAD)

            @pl.when(ci < CHUNKS_PER_TILE - 2)
            def _():
                wait_srcload(ci + 2)
                start_gather(ci + 2, nb)

            wait_gather(ci, b)
            rbuf = rows[b]

            @plsc.parallel_loop(0, CHUNK // 16, unroll=2)
            def grp(g):
                w16 = wbuf[ci, pl.ds(g * 16, 16)]
                for l in range(16):
                    e = g * 16 + l
                    wv = jnp.full((16,), w16[l], jnp.float32)
                    for f in range(DH // 16):
                        sl = pl.ds(f * 16, 16)
                        rbuf[e, sl] = rbuf[e, sl] * wv

            start_scatter(ci, b)

        def outer(oi, carry):
            step(oi * 4, 0, 2)
            step(oi * 4 + 1, 1, 3)
            step(oi * 4 + 2, 2, 0)
            step(oi * 4 + 3, 3, 1)
            return carry

        lax.fori_loop(0, CHUNKS_PER_TILE // 4, outer, 0)
        wait_scatter(CHUNKS_PER_TILE - 2, (CHUNKS_PER_TILE - 2) % 4)
        wait_scatter(CHUNKS_PER_TILE - 1, (CHUNKS_PER_TILE - 1) % 4)
        plsc.subcore_barrier()

        sl = pl.ds(s * ROWS_PER_TILE, ROWS_PER_TILE)
        pltpu.sync_copy(acc_sh.at[sl], out.at[sl])
        plsc.subcore_barrier()


@jax.jit
def _propagate(src, dst, w, ini_st):
    mesh = plsc.VectorSubcoreMesh(core_axis_name="c", subcore_axis_name="s")
    gcn = pl.kernel(
        _gcn_body,
        out_type=(
            jax.ShapeDtypeStruct((NC, N_PAD, DH), jnp.float32),
            jax.ShapeDtypeStruct((NC, N_PAD, DH), jnp.float32),
            jax.ShapeDtypeStruct((NC, N_PAD, DH), jnp.float32),
        ),
        mesh=mesh,
        compiler_params=pltpu.CompilerParams(use_tc_tiling_on_sc=False),
        scratch_types=[
            pltpu.VMEM_SHARED((N_PAD, DH), jnp.float32),
            pltpu.VMEM((CHUNKS_PER_TILE, CHUNK), jnp.int32),
            pltpu.VMEM((CHUNKS_PER_TILE, CHUNK), jnp.float32),
            pltpu.VMEM((CHUNK, DH), jnp.float32),
            pltpu.VMEM((CHUNK, DH), jnp.float32),
            pltpu.VMEM((CHUNK, DH), jnp.float32),
            pltpu.VMEM((CHUNK, DH), jnp.float32),
            pltpu.VMEM((SRCN, CHUNK), jnp.int32),
            pltpu.SemaphoreType.DMA,
            pltpu.SemaphoreType.DMA,
            pltpu.SemaphoreType.DMA,
            pltpu.SemaphoreType.DMA,
            pltpu.SemaphoreType.DMA,
            pltpu.SemaphoreType.DMA,
            pltpu.SemaphoreType.DMA,
            pltpu.SemaphoreType.DMA,
            pltpu.SemaphoreType.DMA((SRCN,)),
        ],
    )
    o1, o2, o3 = gcn(src, dst, w, ini_st)
    total = ini_st + o1 + o2 + o3
    sum_emb = jnp.concatenate([total[0], total[1]], axis=1) / (LAYERS + 1)
    return sum_emb


def kernel(adj_edge_index, adj_edge_weight, subseq_emb, target_emb):
    src = adj_edge_index[0]
    dst = adj_edge_index[1]
    pad = E_PAD - N_EDGES_RAW
    pad_idx = (jnp.arange(pad, dtype=jnp.int32) * 37) % N_NODES
    src = jnp.concatenate([src, pad_idx]).reshape(TOTAL_CHUNKS, CHUNK)
    dst = jnp.concatenate([dst, pad_idx]).reshape(TOTAL_CHUNKS, CHUNK)
    w = jnp.concatenate(
        [adj_edge_weight, jnp.zeros((pad,), jnp.float32)]
    ).reshape(TOTAL_CHUNKS, CHUNK)
    ini = jnp.concatenate(
        [subseq_emb, target_emb, jnp.zeros((N_PAD - N_NODES, D), jnp.float32)],
        axis=0)
    ini_st = jnp.stack([ini[:, :DH], ini[:, DH:]], axis=0)
    sum_emb = _propagate(src, dst, w, ini_st)[:N_NODES]
    return (sum_emb[:N_SUBSEQ], sum_emb[N_SUBSEQ:])


# speedup vs baseline: 12.3656x; 1.0762x over previous
import jax
import jax.numpy as jnp
from jax import lax
from jax.experimental import pallas as pl
from jax.experimental.pallas import tpu as pltpu
from jax.experimental.pallas import tpu_sc as plsc

N_NODES = 10000
N_PAD = 10240
N_SUBSEQ = 6000
D = 128
DH = D // 2
N_EDGES_RAW = 320000
LAYERS = 3

NC = 2
NS = 16
CHUNK = 128
CHUNKS_PER_TILE = 160
EDGES_PER_TILE = CHUNK * CHUNKS_PER_TILE
E_PAD = EDGES_PER_TILE * NS
TOTAL_CHUNKS = E_PAD // CHUNK

ROWS_PER_TILE = N_PAD // NS
WB = 128
WB_STEPS = ROWS_PER_TILE // WB


SRCN = 8
SRC_AHEAD = 5


def _gcn_body(src_hbm, dst_hbm, w_hbm, ini_st, o1_st, o2_st,
              out1_hbm, out2_hbm,
              acc_sh, dstb, wbuf, rows0, rows1, rows2, rows3, sbufs,
              gsem0, gsem1, gsem2, gsem3, ssem0, ssem1, ssem2, ssem3,
              srcsems):
    c = lax.axis_index("c")
    s = lax.axis_index("s")
    rows = (rows0, rows1, rows2, rows3)
    gsem = (gsem0, gsem1, gsem2, gsem3)
    ssem = (ssem0, ssem1, ssem2, ssem3)
    tables = (ini_st, o1_st, o2_st)
    outs = (o1_st, o2_st)

    chunk0 = s * CHUNKS_PER_TILE
    pltpu.sync_copy(dst_hbm.at[pl.ds(chunk0, CHUNKS_PER_TILE)], dstb)
    pltpu.sync_copy(w_hbm.at[pl.ds(chunk0, CHUNKS_PER_TILE)], wbuf)

    zero16 = jnp.zeros((16,), jnp.float32)

    for layer in range(LAYERS):
        tbl = tables[layer].at[c]

        def zrow(r, carry):
            for f in range(DH // 16):
                rows0[r, pl.ds(f * 16, 16)] = zero16
            return carry

        lax.fori_loop(0, WB, zrow, 0)
        for b in range(WB_STEPS):
            pltpu.sync_copy(
                rows0, acc_sh.at[pl.ds(s * ROWS_PER_TILE + b * WB, WB)])
        plsc.subcore_barrier()

        def start_scatter(ci, b):
            pltpu.async_copy(rows[b], acc_sh.at[dstb.at[ci]], ssem[b], add=True)

        def wait_scatter(ci, b):
            pltpu.make_async_copy(rows[b], acc_sh.at[dstb.at[ci]], ssem[b]).wait()

        def start_srcload(ci):
            j = ci % SRCN
            pltpu.async_copy(src_hbm.at[chunk0 + ci], sbufs.at[j],
                             srcsems.at[j])

        def wait_srcload(ci):
            j = ci % SRCN
            pltpu.make_async_copy(src_hbm.at[chunk0 + ci], sbufs.at[j],
                                  srcsems.at[j]).wait()

        def start_gather(ci, b):
            pltpu.async_copy(tbl.at[sbufs.at[ci % SRCN]], rows[b], gsem[b])

        def wait_gather(ci, b):
            pltpu.make_async_copy(
                tbl.at[sbufs.at[ci % SRCN]], rows[b], gsem[b]).wait()

        for j in range(SRC_AHEAD):
            start_srcload(j)
        wait_srcload(0)
        start_gather(0, 0)
        wait_srcload(1)
        start_gather(1, 1)

        def step(ci, b, nb):
            @pl.when(ci >= 2)
            def _():
                wait_scatter(ci - 2, nb)

            @pl.when(ci < CHUNKS_PER_TILE - SRC_AHEAD)
            def _():
                start_srcload(ci + SRC_AHEAD)

            @pl.when(ci < CHUNKS_PER_TILE - 2)
            def _():
                wait_srcload(ci + 2)
                start_gather(ci + 2, nb)

            wait_gather(ci, b)
            rbuf = rows[b]

            @plsc.parallel_loop(0, CHUNK // 16, unroll=2)
            def grp(g):
                w16 = wbuf[ci, pl.ds(g * 16, 16)]
                for l in range(16):
                    e = g * 16 + l
                    wv = jnp.full((16,), w16[l], jnp.float32)
                    for f in range(DH // 16):
                        sl = pl.ds(f * 16, 16)
                        rbuf[e, sl] = rbuf[e, sl] * wv

            start_scatter(ci, b)

        def outer(oi, carry):
            step(oi * 4, 0, 2)
            step(oi * 4 + 1, 1, 3)
            step(oi * 4 + 2, 2, 0)
            step(oi * 4 + 3, 3, 1)
            return carry

        lax.fori_loop(0, CHUNKS_PER_TILE // 4, outer, 0)
        wait_scatter(CHUNKS_PER_TILE - 2, (CHUNKS_PER_TILE - 2) % 4)
        wait_scatter(CHUNKS_PER_TILE - 1, (CHUNKS_PER_TILE - 1) % 4)
        plsc.subcore_barrier()

        if layer < LAYERS - 1:
            out = outs[layer].at[c]
            sl = pl.ds(s * ROWS_PER_TILE, ROWS_PER_TILE)
            pltpu.sync_copy(acc_sh.at[sl], out.at[sl])
            plsc.subcore_barrier()
        else:
            colsl = pl.ds(c * DH, DH)
            for wb in range(WB_STEPS):
                r0 = s * ROWS_PER_TILE + wb * WB
                bsl = pl.ds(r0, WB)
                pltpu.sync_copy(acc_sh.at[bsl], rows0)
                pltpu.sync_copy(ini_st.at[c].at[bsl], rows1)
                pltpu.sync_copy(o1_st.at[c].at[bsl], rows2)
                pltpu.sync_copy(o2_st.at[c].at[bsl], rows3)

                @plsc.parallel_loop(0, WB, unroll=2)
                def crow(r):
                    for f in range(DH // 16):
                        fl = pl.ds(f * 16, 16)
                        rows0[r, fl] = (rows0[r, fl] + rows1[r, fl]
                                        + rows2[r, fl] + rows3[r, fl]) * 0.25

                @pl.when(r0 + WB <= N_SUBSEQ)
                def _():
                    pltpu.sync_copy(rows0, out1_hbm.at[pl.ds(r0, WB), colsl])

                @pl.when(jnp.logical_and(r0 < N_SUBSEQ, r0 + WB > N_SUBSEQ))
                def _():
                    pltpu.sync_copy(rows0.at[pl.ds(0, 112)],
                                    out1_hbm.at[pl.ds(r0, 112), colsl])
                    pltpu.sync_copy(rows0.at[pl.ds(112, 16)],
                                    out2_hbm.at[pl.ds(0, 16), colsl])

                @pl.when(jnp.logical_and(r0 >= N_SUBSEQ, r0 + WB <= N_NODES))
                def _():
                    pltpu.sync_copy(
                        rows0, out2_hbm.at[pl.ds(r0 - N_SUBSEQ, WB), colsl])

                @pl.when(jnp.logical_and(r0 < N_NODES, r0 + WB > N_NODES))
                def _():
                    pltpu.sync_copy(rows0.at[pl.ds(0, 16)],
                                    out2_hbm.at[pl.ds(r0 - N_SUBSEQ, 16), colsl])


@jax.jit
def _propagate(src, dst, w, ini_st):
    mesh = plsc.VectorSubcoreMesh(core_axis_name="c", subcore_axis_name="s")
    gcn = pl.kernel(
        _gcn_body,
        out_type=(
            jax.ShapeDtypeStruct((NC, N_PAD, DH), jnp.float32),
            jax.ShapeDtypeStruct((NC, N_PAD, DH), jnp.float32),
            jax.ShapeDtypeStruct((N_SUBSEQ, D), jnp.float32),
            jax.ShapeDtypeStruct((N_NODES - N_SUBSEQ, D), jnp.float32),
        ),
        mesh=mesh,
        compiler_params=pltpu.CompilerParams(use_tc_tiling_on_sc=False),
        scratch_types=[
            pltpu.VMEM_SHARED((N_PAD, DH), jnp.float32),
            pltpu.VMEM((CHUNKS_PER_TILE, CHUNK), jnp.int32),
            pltpu.VMEM((CHUNKS_PER_TILE, CHUNK), jnp.float32),
            pltpu.VMEM((CHUNK, DH), jnp.float32),
            pltpu.VMEM((CHUNK, DH), jnp.float32),
            pltpu.VMEM((CHUNK, DH), jnp.float32),
            pltpu.VMEM((CHUNK, DH), jnp.float32),
            pltpu.VMEM((SRCN, CHUNK), jnp.int32),
            pltpu.SemaphoreType.DMA,
            pltpu.SemaphoreType.DMA,
            pltpu.SemaphoreType.DMA,
            pltpu.SemaphoreType.DMA,
            pltpu.SemaphoreType.DMA,
            pltpu.SemaphoreType.DMA,
            pltpu.SemaphoreType.DMA,
            pltpu.SemaphoreType.DMA,
            pltpu.SemaphoreType.DMA((SRCN,)),
        ],
    )
    o1, o2, out_s, out_t = gcn(src, dst, w, ini_st)
    return out_s, out_t


def kernel(adj_edge_index, adj_edge_weight, subseq_emb, target_emb):
    src = adj_edge_index[0]
    dst = adj_edge_index[1]
    pad = E_PAD - N_EDGES_RAW
    pad_idx = (jnp.arange(pad, dtype=jnp.int32) * 37) % N_NODES
    src = jnp.concatenate([src, pad_idx]).reshape(TOTAL_CHUNKS, CHUNK)
    dst = jnp.concatenate([dst, pad_idx]).reshape(TOTAL_CHUNKS, CHUNK)
    w = jnp.concatenate(
        [adj_edge_weight, jnp.zeros((pad,), jnp.float32)]
    ).reshape(TOTAL_CHUNKS, CHUNK)
    ini = jnp.concatenate(
        [subseq_emb, target_emb, jnp.zeros((N_PAD - N_NODES, D), jnp.float32)],
        axis=0)
    ini_st = jnp.stack([ini[:, :DH], ini[:, DH:]], axis=0)
    return _propagate(src, dst, w, ini_st)
